# baseline (speedup 1.0000x reference)
import functools
import numpy as np
import jax
import jax.numpy as jnp
from jax import lax
from jax.experimental import pallas as pl
from jax.experimental.pallas import tpu as pltpu
from jax.experimental.pallas import tpu_sc as plsc

B = 16384
ND = 13
NF = 26
D = 32
V1 = 100001
NROW = 39

NW = 32
RW = B // NW
CB = 64
NCHUNK = RW // CB
E = CB * NF
NS_G = E // 128
TD = CB * ND
NS_D = TD // 64


def _sc_assemble(tab, spx, dens3, goffs, opat, dpat):
    mesh = plsc.VectorSubcoreMesh(core_axis_name="c", subcore_axis_name="s")

    @functools.partial(
        pl.kernel,
        out_type=jax.ShapeDtypeStruct((B * NROW, D), jnp.float32),
        mesh=mesh,
        scratch_types=[
            pltpu.VMEM((E,), jnp.int32),
            pltpu.VMEM((E,), jnp.int32),
            pltpu.VMEM((E,), jnp.int32),
            pltpu.VMEM((E,), jnp.int32),
            pltpu.VMEM((TD,), jnp.int32),
            pltpu.VMEM((NS_G, 128), jnp.int32),
            pltpu.VMEM((NS_D, 64), jnp.int32),
            pltpu.VMEM((NS_G, 128, D), jnp.float32),
            pltpu.VMEM((NS_D, 64, D), jnp.float32),
            pltpu.SemaphoreType.DMA,
            pltpu.SemaphoreType.DMA,
        ],
        compiler_params=pltpu.CompilerParams(use_tc_tiling_on_sc=False),
    )
    def k(tab_h, spx_h, dens_h, goffs_h, opat_h, dpat_h, out_h,
          sidx, gidx, goffs_v, opat_v, dpat_v, oidx_v, didx_v,
          rows, dbuf, gsem, ssem):
        wid = lax.axis_index("s") * 2 + lax.axis_index("c")
        pltpu.sync_copy(goffs_h, goffs_v)
        pltpu.sync_copy(opat_h, opat_v)
        pltpu.sync_copy(dpat_h, dpat_v)

        def chunk(c, _):
            b0 = wid * RW + c * CB
            e0 = b0 * NF
            rd0 = (b0 * ND) // 64
            base = b0 * NROW

            pltpu.sync_copy(spx_h.at[pl.ds(e0, E)], sidx)
            pltpu.sync_copy(dens_h.at[pl.ds(rd0, NS_D)], dbuf)

            def dbody(i, _):
                sl = pl.ds(i * 16, 16)
                didx_v[i // 4, pl.ds((i % 4) * 16, 16)] = dpat_v[sl] + base
                return _
            lax.fori_loop(0, TD // 16, dbody, None)
            dsc = [
                pltpu.async_copy(dbuf.at[j], out_h.at[didx_v.at[j]], ssem)
                for j in range(NS_D)
            ]

            def gbody(i, _):
                sl = pl.ds(i * 16, 16)
                gidx[sl] = sidx[sl] + goffs_v[sl]
                oidx_v[i // 8, pl.ds((i % 8) * 16, 16)] = opat_v[sl] + base
                return _
            lax.fori_loop(0, E // 16, gbody, None)

            gth = [
                pltpu.async_copy(tab_h.at[gidx.at[pl.ds(j * 128, 128)]],
                                 rows.at[j], gsem)
                for j in range(NS_G)
            ]
            for d in gth:
                d.wait()

            ssc = [
                pltpu.async_copy(rows.at[j], out_h.at[oidx_v.at[j]], ssem)
                for j in range(NS_G)
            ]
            for d in dsc:
                d.wait()
            for d in ssc:
                d.wait()
            return _

        lax.fori_loop(0, NCHUNK, chunk, None)

    return k(tab, spx, dens3, goffs, opat, dpat)


def _dense_matmul(dense_x, W_dense, b_dense):
    bb = 1024

    def mm(x_ref, w_ref, b_ref, o_ref):
        o_ref[...] = (
            jnp.dot(x_ref[...], w_ref[...], preferred_element_type=jnp.float32)
            + b_ref[...]
        )

    return pl.pallas_call(
        mm,
        grid=(B // bb,),
        in_specs=[
            pl.BlockSpec((bb, ND), lambda i: (i, 0)),
            pl.BlockSpec((ND, ND * D), lambda i: (0, 0)),
            pl.BlockSpec((1, ND * D), lambda i: (0, 0)),
        ],
        out_specs=pl.BlockSpec((bb, ND * D), lambda i: (i, 0)),
        out_shape=jax.ShapeDtypeStruct((B, ND * D), jnp.float32),
    )(dense_x, W_dense, b_dense.reshape(1, ND * D))


def kernel(dense_x, sparse_x, W_dense, b_dense, tables):
    t = np.arange(E, dtype=np.int64)
    goffs = ((t % NF) * V1).astype(np.int32)
    opat = (t + 13 * (t // NF) + ND).astype(np.int32)
    td = np.arange(TD, dtype=np.int64)
    dpat = (td + NF * (td // ND)).astype(np.int32)

    dens = _dense_matmul(dense_x, W_dense, b_dense)
    out = _sc_assemble(
        tables.reshape(NF * V1, D),
        sparse_x.reshape(-1),
        dens.reshape(-1, 64, D),
        jnp.asarray(goffs),
        jnp.asarray(opat),
        jnp.asarray(dpat),
    )
    return out.reshape(B, NROW, D)
